# Initial kernel scaffold; baseline (speedup 1.0000x reference)
import jax
import jax.numpy as jnp
from jax.experimental import pallas as pl


def kernel(node_feat, edge_feat, edge_index, graph_ids, Wn, bn, We, be, Wc0, bc0, Wc1, bc1, Wc2, bc2, Wfp, bfp):
    raise NotImplementedError("write your pallas kernel here")


# speedup vs baseline: 3.3016x; 3.3016x over previous
import jax
import jax.numpy as jnp
from jax import lax
from jax.experimental import pallas as pl
from jax.experimental.pallas import tpu as pltpu
from jax.experimental.pallas import tpu_sc as plsc

N = 10000
E = 320000
D = 128
G = 64

NC = 2
NS = 16
CHUNK = 128
N_CHUNKS = 2560
E_PAD = N_CHUNKS * CHUNK
N_PAD = 10240
BR = 256
EFW = 24
HW = D // 2
GRP = 16

_SC_PARAMS = pltpu.CompilerParams(use_tc_tiling_on_sc=False)

_sc_mesh = plsc.VectorSubcoreMesh(
    core_axis_name="c", subcore_axis_name="s", num_cores=NC, num_subcores=NS)


def _zero_buf(buf_ref, width):
  z = jnp.zeros((16,), jnp.float32)

  @pl.loop(0, CHUNK)
  def _(r):
    for i in range(width // 16):
      buf_ref[r, pl.ds(i * 16, 16)] = z


def _node_pool_body(cur2, src_hbm, dst_hbm, out_hbm, srcv, dstv, rows, acc,
                    gsem):
  cid = lax.axis_index("c")
  sid = lax.axis_index("s")
  rows_per_tile = N_PAD // NS
  data = cur2.at[cid]

  _zero_buf(rows.at[0], HW)
  for j in range(rows_per_tile // CHUNK):
    pltpu.sync_copy(rows.at[0],
                    acc.at[pl.ds(sid * rows_per_tile + j * CHUNK, CHUNK)])
  plsc.subcore_barrier()

  cps = N_CHUNKS // NS

  def start(c, buf):
    return pltpu.async_copy(data.at[srcv.at[c]], rows.at[buf], gsem)

  @pl.loop(0, cps // GRP)
  def _(g):
    base = sid * cps + g * GRP
    pltpu.sync_copy(src_hbm.at[pl.ds(base, GRP)], srcv)
    pltpu.sync_copy(dst_hbm.at[pl.ds(base, GRP)], dstv)
    start(0, 0)

    @pl.loop(0, GRP)
    def _(c):
      buf = lax.rem(c, 2)

      @pl.when(c + 1 < GRP)
      def _():
        start(c + 1, 1 - buf)

      pltpu.make_async_copy(data.at[pl.ds(0, CHUNK)], rows.at[buf],
                            gsem).wait()
      pltpu.sync_copy(rows.at[buf], acc.at[dstv.at[c]], add=True)

  plsc.subcore_barrier()
  pltpu.sync_copy(acc.at[pl.ds(sid * rows_per_tile, rows_per_tile)],
                  out_hbm.at[cid].at[pl.ds(sid * rows_per_tile,
                                           rows_per_tile)])


_sc_node_pool = pl.kernel(
    _node_pool_body,
    out_type=jax.ShapeDtypeStruct((NC, N_PAD, HW), jnp.float32),
    mesh=_sc_mesh,
    scratch_types=[
        pltpu.VMEM((GRP, CHUNK), jnp.int32),
        pltpu.VMEM((GRP, CHUNK), jnp.int32),
        pltpu.VMEM((2, CHUNK, HW), jnp.float32),
        pltpu.VMEM_SHARED((N_PAD, HW), jnp.float32),
        pltpu.SemaphoreType.DMA,
    ],
    compiler_params=_SC_PARAMS,
    name="sc_node_pool",
)


def _edge_pool_body(ef_hbm, dst_hbm, out_hbm, dstv, rows, acc, gsem):
  cid = lax.axis_index("c")
  sid = lax.axis_index("s")
  wid = cid * NS + sid
  rows_per_tile = N_PAD // NS
  cpw = N_CHUNKS // (NC * NS)

  _zero_buf(rows.at[0], EFW)
  for j in range(rows_per_tile // CHUNK):
    pltpu.sync_copy(rows.at[0],
                    acc.at[pl.ds(sid * rows_per_tile + j * CHUNK, CHUNK)])
  plsc.subcore_barrier()

  base = wid * cpw
  pltpu.sync_copy(dst_hbm.at[pl.ds(base, cpw)], dstv)

  def start(c, buf):
    return pltpu.async_copy(ef_hbm.at[pl.ds((base + c) * CHUNK, CHUNK)],
                            rows.at[buf], gsem)

  start(0, 0)

  @pl.loop(0, cpw)
  def _(c):
    buf = lax.rem(c, 2)

    @pl.when(c + 1 < cpw)
    def _():
      start(c + 1, 1 - buf)

    pltpu.make_async_copy(ef_hbm.at[pl.ds(0, CHUNK)], rows.at[buf],
                          gsem).wait()
    pltpu.sync_copy(rows.at[buf], acc.at[dstv.at[c]], add=True)

  plsc.subcore_barrier()
  pltpu.sync_copy(acc.at[pl.ds(sid * rows_per_tile, rows_per_tile)],
                  out_hbm.at[cid].at[pl.ds(sid * rows_per_tile,
                                           rows_per_tile)])


_sc_edge_pool = pl.kernel(
    _edge_pool_body,
    out_type=jax.ShapeDtypeStruct((NC, N_PAD, EFW), jnp.float32),
    mesh=_sc_mesh,
    scratch_types=[
        pltpu.VMEM((N_CHUNKS // (NC * NS), CHUNK), jnp.int32),
        pltpu.VMEM((2, CHUNK, EFW), jnp.float32),
        pltpu.VMEM_SHARED((N_PAD, EFW), jnp.float32),
        pltpu.SemaphoreType.DMA,
    ],
    compiler_params=_SC_PARAMS,
    name="sc_edge_pool",
)


def _tc_msg_body(nf, efp, Wn, We2, bn, msg_ref, cur_ref):
  m = jnp.dot(nf[...], Wn[...], preferred_element_type=jnp.float32)
  m += jnp.dot(efp[0] + efp[1], We2[...], preferred_element_type=jnp.float32)
  m += bn[...]
  msg_ref[...] = m
  r = jnp.maximum(m, 0.0)
  cur_ref[0] = r[:, :HW]
  cur_ref[1] = r[:, HW:]


def _tc_msg(nf, efp, Wn, We2, bn):
  return pl.pallas_call(
      _tc_msg_body,
      grid=(N_PAD // BR,),
      in_specs=[
          pl.BlockSpec((BR, D), lambda i: (i, 0)),
          pl.BlockSpec((NC, BR, EFW), lambda i: (0, i, 0)),
          pl.BlockSpec((D, D), lambda i: (0, 0)),
          pl.BlockSpec((EFW, D), lambda i: (0, 0)),
          pl.BlockSpec((1, D), lambda i: (0, 0)),
      ],
      out_specs=[pl.BlockSpec((BR, D), lambda i: (i, 0)),
                 pl.BlockSpec((NC, BR, HW), lambda i: (0, i, 0))],
      out_shape=[jax.ShapeDtypeStruct((N_PAD, D), jnp.float32),
                 jax.ShapeDtypeStruct((NC, N_PAD, HW), jnp.float32)],
  )(nf, efp, Wn, We2, bn)


def _tc_round_body(npool, msg, Wc, bc, cur_ref):
  s = jnp.concatenate([npool[0], npool[1]], axis=1)
  m = jnp.dot(s, Wc[...], preferred_element_type=jnp.float32)
  m += bc[...] + msg[...]
  r = jnp.maximum(m, 0.0)
  cur_ref[0] = r[:, :HW]
  cur_ref[1] = r[:, HW:]


def _tc_round(npool, msg, Wc, bc):
  return pl.pallas_call(
      _tc_round_body,
      grid=(N_PAD // BR,),
      in_specs=[
          pl.BlockSpec((NC, BR, HW), lambda i: (0, i, 0)),
          pl.BlockSpec((BR, D), lambda i: (i, 0)),
          pl.BlockSpec((D, D), lambda i: (0, 0)),
          pl.BlockSpec((1, D), lambda i: (0, 0)),
      ],
      out_specs=pl.BlockSpec((NC, BR, HW), lambda i: (0, i, 0)),
      out_shape=jax.ShapeDtypeStruct((NC, N_PAD, HW), jnp.float32),
  )(npool, msg, Wc, bc)


def _tc_pool_body(cur2, gid, Wfp, bfp, y_ref):
  i = pl.program_id(0)
  x = jnp.concatenate([cur2[0], cur2[1]], axis=1)
  x = jnp.dot(x, Wfp[...], preferred_element_type=jnp.float32)
  x = jnp.maximum(x + bfp[...], 0.0)
  ids = gid[0, 0, :]
  oh = (lax.broadcasted_iota(jnp.int32, (G, BR), 0) == ids[None, :])
  contrib = jnp.dot(oh.astype(jnp.float32), x,
                    preferred_element_type=jnp.float32)

  @pl.when(i == 0)
  def _():
    y_ref[...] = jnp.zeros_like(y_ref)

  y_ref[...] += contrib


def _tc_pool(cur2, gid3, Wfp, bfp):
  return pl.pallas_call(
      _tc_pool_body,
      grid=(N_PAD // BR,),
      in_specs=[
          pl.BlockSpec((NC, BR, HW), lambda i: (0, i, 0)),
          pl.BlockSpec((1, 1, BR), lambda i: (i, 0, 0)),
          pl.BlockSpec((D, D), lambda i: (0, 0)),
          pl.BlockSpec((1, D), lambda i: (0, 0)),
      ],
      out_specs=pl.BlockSpec((G, D), lambda i: (0, 0)),
      out_shape=jax.ShapeDtypeStruct((G, D), jnp.float32),
  )(cur2, gid3, Wfp, bfp)


def kernel(node_feat, edge_feat, edge_index, graph_ids, Wn, bn, We, be,
           Wc0, bc0, Wc1, bc1, Wc2, bc2, Wfp, bfp):
  src = edge_index[0]
  dst = edge_index[1]
  pad_e = E_PAD - E
  pad_n = N_PAD - N

  ef24 = jnp.concatenate(
      [edge_feat, jnp.ones((E, 1), jnp.float32),
       jnp.zeros((E, EFW - edge_feat.shape[1] - 1), jnp.float32)], axis=1)
  ef24 = jnp.concatenate([ef24, jnp.zeros((pad_e, EFW), jnp.float32)], axis=0)
  src_p = jnp.concatenate(
      [src, jnp.zeros((pad_e,), jnp.int32)]).reshape(N_CHUNKS, CHUNK)
  dst_p = jnp.concatenate(
      [dst, jnp.full((pad_e,), N_PAD - 1, jnp.int32)]).reshape(N_CHUNKS, CHUNK)
  nf_p = jnp.concatenate([node_feat, jnp.zeros((pad_n, D), jnp.float32)])
  gid3 = jnp.concatenate(
      [graph_ids, jnp.full((pad_n,), G, jnp.int32)]).reshape(N_PAD // BR, 1, BR)
  We2 = jnp.concatenate(
      [We, be[None, :], jnp.zeros((EFW - We.shape[0] - 1, D), jnp.float32)],
      axis=0)
  Wc_all = jnp.stack([Wc0, Wc1, Wc2])
  bc_all = jnp.stack([bc0, bc1, bc2]).reshape(3, 1, D)

  efpool = _sc_edge_pool(ef24, dst_p)
  msg, cur2 = _tc_msg(nf_p, efpool, Wn, We2, bn.reshape(1, D))

  def round_body(r, cur2):
    npool = _sc_node_pool(cur2, src_p, dst_p)
    return _tc_round(npool, msg, Wc_all[r], bc_all[r])

  cur2 = lax.fori_loop(0, 3, round_body, cur2)
  return _tc_pool(cur2, gid3, Wfp, bfp.reshape(1, D))


# speedup vs baseline: 3.4904x; 1.0572x over previous
import jax
import jax.numpy as jnp
from jax import lax
from jax.experimental import pallas as pl
from jax.experimental.pallas import tpu as pltpu
from jax.experimental.pallas import tpu_sc as plsc

N = 10000
E = 320000
D = 128
G = 64

NC = 2
NS = 16
CHUNK = 128
N_CHUNKS = 2560
E_PAD = N_CHUNKS * CHUNK
N_PAD = 10240
BR = 256
EFW = 24
HW = D // 2
NBUF = 6

_SC_PARAMS = pltpu.CompilerParams(use_tc_tiling_on_sc=False)

_sc_mesh = plsc.VectorSubcoreMesh(
    core_axis_name="c", subcore_axis_name="s", num_cores=NC, num_subcores=NS)


def _zero_buf(buf_ref, width):
  z = jnp.zeros((16,), jnp.float32)

  @pl.loop(0, CHUNK)
  def _(r):
    for i in range(width // 16):
      buf_ref[r, pl.ds(i * 16, 16)] = z


def _run_ring(cps, start, wait_gather, rows, acc, dstv, ssem):
  for b in range(4):
    start(b, b)

  def wait_one_scatter():
    pltpu.make_async_copy(rows.at[0], acc.at[dstv.at[0]], ssem).wait()

  @pl.loop(0, cps)
  def _(c):
    buf = lax.rem(c, NBUF)
    wait_gather(buf)
    pltpu.async_copy(rows.at[buf], acc.at[dstv.at[c]], ssem, add=True)

    @pl.when(jnp.logical_and(c >= 2, c + 4 < cps))
    def _():
      wait_one_scatter()

    @pl.when(c + 4 < cps)
    def _():
      start(c + 4, lax.rem(c + 4, NBUF))

  for _ in range(NBUF):
    wait_one_scatter()


def _node_pool_body(cur2, src_hbm, dst_hbm, out_hbm, srcv, dstv, rows, acc,
                    gsem, ssem):
  cid = lax.axis_index("c")
  sid = lax.axis_index("s")
  rows_per_tile = N_PAD // NS
  data = cur2.at[cid]

  _zero_buf(rows.at[0], HW)
  for j in range(rows_per_tile // CHUNK):
    pltpu.sync_copy(rows.at[0],
                    acc.at[pl.ds(sid * rows_per_tile + j * CHUNK, CHUNK)])
  plsc.subcore_barrier()

  cps = N_CHUNKS // NS
  base = sid * cps
  pltpu.sync_copy(src_hbm.at[pl.ds(base, cps)], srcv)
  pltpu.sync_copy(dst_hbm.at[pl.ds(base, cps)], dstv)

  def start(c, buf):
    pltpu.async_copy(data.at[srcv.at[c]], rows.at[buf], gsem)

  def wait_gather(buf):
    pltpu.make_async_copy(data.at[pl.ds(0, CHUNK)], rows.at[buf], gsem).wait()

  _run_ring(cps, start, wait_gather, rows, acc, dstv, ssem)

  plsc.subcore_barrier()
  pltpu.sync_copy(acc.at[pl.ds(sid * rows_per_tile, rows_per_tile)],
                  out_hbm.at[cid].at[pl.ds(sid * rows_per_tile,
                                           rows_per_tile)])


_sc_node_pool = pl.kernel(
    _node_pool_body,
    out_type=jax.ShapeDtypeStruct((NC, N_PAD, HW), jnp.float32),
    mesh=_sc_mesh,
    scratch_types=[
        pltpu.VMEM((N_CHUNKS // NS, CHUNK), jnp.int32),
        pltpu.VMEM((N_CHUNKS // NS, CHUNK), jnp.int32),
        pltpu.VMEM((NBUF, CHUNK, HW), jnp.float32),
        pltpu.VMEM_SHARED((N_PAD, HW), jnp.float32),
        pltpu.SemaphoreType.DMA,
        pltpu.SemaphoreType.DMA,
    ],
    compiler_params=_SC_PARAMS,
    name="sc_node_pool",
)


def _edge_pool_body(ef_hbm, dst_hbm, out_hbm, dstv, rows, acc, gsem, ssem):
  cid = lax.axis_index("c")
  sid = lax.axis_index("s")
  wid = cid * NS + sid
  rows_per_tile = N_PAD // NS
  cpw = N_CHUNKS // (NC * NS)

  _zero_buf(rows.at[0], EFW)
  for j in range(rows_per_tile // CHUNK):
    pltpu.sync_copy(rows.at[0],
                    acc.at[pl.ds(sid * rows_per_tile + j * CHUNK, CHUNK)])
  plsc.subcore_barrier()

  base = wid * cpw
  pltpu.sync_copy(dst_hbm.at[pl.ds(base, cpw)], dstv)

  def start(c, buf):
    pltpu.async_copy(ef_hbm.at[pl.ds((base + c) * CHUNK, CHUNK)],
                     rows.at[buf], gsem)

  def wait_gather(buf):
    pltpu.make_async_copy(ef_hbm.at[pl.ds(0, CHUNK)], rows.at[buf],
                          gsem).wait()

  _run_ring(cpw, start, wait_gather, rows, acc, dstv, ssem)

  plsc.subcore_barrier()
  pltpu.sync_copy(acc.at[pl.ds(sid * rows_per_tile, rows_per_tile)],
                  out_hbm.at[cid].at[pl.ds(sid * rows_per_tile,
                                           rows_per_tile)])


_sc_edge_pool = pl.kernel(
    _edge_pool_body,
    out_type=jax.ShapeDtypeStruct((NC, N_PAD, EFW), jnp.float32),
    mesh=_sc_mesh,
    scratch_types=[
        pltpu.VMEM((N_CHUNKS // (NC * NS), CHUNK), jnp.int32),
        pltpu.VMEM((NBUF, CHUNK, EFW), jnp.float32),
        pltpu.VMEM_SHARED((N_PAD, EFW), jnp.float32),
        pltpu.SemaphoreType.DMA,
        pltpu.SemaphoreType.DMA,
    ],
    compiler_params=_SC_PARAMS,
    name="sc_edge_pool",
)


def _tc_msg_body(nf, efp, Wn, We2, bn, msg_ref, cur_ref):
  m = jnp.dot(nf[...], Wn[...], preferred_element_type=jnp.float32)
  m += jnp.dot(efp[0] + efp[1], We2[...], preferred_element_type=jnp.float32)
  m += bn[...]
  msg_ref[...] = m
  r = jnp.maximum(m, 0.0)
  cur_ref[0] = r[:, :HW]
  cur_ref[1] = r[:, HW:]


def _tc_msg(nf, efp, Wn, We2, bn):
  return pl.pallas_call(
      _tc_msg_body,
      grid=(N_PAD // BR,),
      in_specs=[
          pl.BlockSpec((BR, D), lambda i: (i, 0)),
          pl.BlockSpec((NC, BR, EFW), lambda i: (0, i, 0)),
          pl.BlockSpec((D, D), lambda i: (0, 0)),
          pl.BlockSpec((EFW, D), lambda i: (0, 0)),
          pl.BlockSpec((1, D), lambda i: (0, 0)),
      ],
      out_specs=[pl.BlockSpec((BR, D), lambda i: (i, 0)),
                 pl.BlockSpec((NC, BR, HW), lambda i: (0, i, 0))],
      out_shape=[jax.ShapeDtypeStruct((N_PAD, D), jnp.float32),
                 jax.ShapeDtypeStruct((NC, N_PAD, HW), jnp.float32)],
  )(nf, efp, Wn, We2, bn)


def _tc_round_body(npool, msg, Wc, bc, cur_ref):
  s = jnp.concatenate([npool[0], npool[1]], axis=1)
  m = jnp.dot(s, Wc[...], preferred_element_type=jnp.float32)
  m += bc[...] + msg[...]
  r = jnp.maximum(m, 0.0)
  cur_ref[0] = r[:, :HW]
  cur_ref[1] = r[:, HW:]


def _tc_round(npool, msg, Wc, bc):
  return pl.pallas_call(
      _tc_round_body,
      grid=(N_PAD // BR,),
      in_specs=[
          pl.BlockSpec((NC, BR, HW), lambda i: (0, i, 0)),
          pl.BlockSpec((BR, D), lambda i: (i, 0)),
          pl.BlockSpec((D, D), lambda i: (0, 0)),
          pl.BlockSpec((1, D), lambda i: (0, 0)),
      ],
      out_specs=pl.BlockSpec((NC, BR, HW), lambda i: (0, i, 0)),
      out_shape=jax.ShapeDtypeStruct((NC, N_PAD, HW), jnp.float32),
  )(npool, msg, Wc, bc)


def _tc_pool_body(cur2, gid, Wfp, bfp, y_ref):
  i = pl.program_id(0)
  x = jnp.concatenate([cur2[0], cur2[1]], axis=1)
  x = jnp.dot(x, Wfp[...], preferred_element_type=jnp.float32)
  x = jnp.maximum(x + bfp[...], 0.0)
  ids = gid[0, 0, :]
  oh = (lax.broadcasted_iota(jnp.int32, (G, BR), 0) == ids[None, :])
  contrib = jnp.dot(oh.astype(jnp.float32), x,
                    preferred_element_type=jnp.float32)

  @pl.when(i == 0)
  def _():
    y_ref[...] = jnp.zeros_like(y_ref)

  y_ref[...] += contrib


def _tc_pool(cur2, gid3, Wfp, bfp):
  return pl.pallas_call(
      _tc_pool_body,
      grid=(N_PAD // BR,),
      in_specs=[
          pl.BlockSpec((NC, BR, HW), lambda i: (0, i, 0)),
          pl.BlockSpec((1, 1, BR), lambda i: (i, 0, 0)),
          pl.BlockSpec((D, D), lambda i: (0, 0)),
          pl.BlockSpec((1, D), lambda i: (0, 0)),
      ],
      out_specs=pl.BlockSpec((G, D), lambda i: (0, 0)),
      out_shape=jax.ShapeDtypeStruct((G, D), jnp.float32),
  )(cur2, gid3, Wfp, bfp)


def kernel(node_feat, edge_feat, edge_index, graph_ids, Wn, bn, We, be,
           Wc0, bc0, Wc1, bc1, Wc2, bc2, Wfp, bfp):
  src = edge_index[0]
  dst = edge_index[1]
  pad_e = E_PAD - E
  pad_n = N_PAD - N

  ef24 = jnp.concatenate(
      [edge_feat, jnp.ones((E, 1), jnp.float32),
       jnp.zeros((E, EFW - edge_feat.shape[1] - 1), jnp.float32)], axis=1)
  ef24 = jnp.concatenate([ef24, jnp.zeros((pad_e, EFW), jnp.float32)], axis=0)
  src_p = jnp.concatenate(
      [src, jnp.zeros((pad_e,), jnp.int32)]).reshape(N_CHUNKS, CHUNK)
  dst_p = jnp.concatenate(
      [dst, jnp.full((pad_e,), N_PAD - 1, jnp.int32)]).reshape(N_CHUNKS, CHUNK)
  nf_p = jnp.concatenate([node_feat, jnp.zeros((pad_n, D), jnp.float32)])
  gid3 = jnp.concatenate(
      [graph_ids, jnp.full((pad_n,), G, jnp.int32)]).reshape(N_PAD // BR, 1, BR)
  We2 = jnp.concatenate(
      [We, be[None, :], jnp.zeros((EFW - We.shape[0] - 1, D), jnp.float32)],
      axis=0)
  Wc_all = jnp.stack([Wc0, Wc1, Wc2])
  bc_all = jnp.stack([bc0, bc1, bc2]).reshape(3, 1, D)

  efpool = _sc_edge_pool(ef24, dst_p)
  msg, cur2 = _tc_msg(nf_p, efpool, Wn, We2, bn.reshape(1, D))

  def round_body(r, cur2):
    npool = _sc_node_pool(cur2, src_p, dst_p)
    return _tc_round(npool, msg, Wc_all[r], bc_all[r])

  cur2 = lax.fori_loop(0, 3, round_body, cur2)
  return _tc_pool(cur2, gid3, Wfp, bfp.reshape(1, D))
